# baseline (speedup 1.0000x reference)
import jax
import jax.numpy as jnp
from jax import lax
from jax.experimental import pallas as pl
from jax.experimental.pallas import tpu as pltpu
from jax.experimental.pallas import tpu_sc as plsc


_COLS_PER_BLOCK = 1024
_NC, _NS, _L = 2, 16, 16
_NW = _NC * _NS


def _gather_body(table_hbm, labels_hbm, out_hbm, lab_v, idx_v, rows_v, out_v, sem):
    b_per_w = lab_v.shape[0]
    wid = lax.axis_index("s") * _NC + lax.axis_index("c")
    base = wid * b_per_w
    pltpu.sync_copy(labels_hbm.at[pl.ds(base, b_per_w)], lab_v)
    iota = lax.iota(jnp.int32, _L)

    def addr_body(t, c):
        lv = lab_v[pl.ds(t * _L, _L)]
        cv = (
            lax.shift_left(lax.shift_right_logical(lv, 3), 10)
            + ((base + _L * t) // 128) * 8
            + lax.bitwise_and(lv, 7)
        )
        idx_v[pl.ds(t * _L, _L)] = jnp.broadcast_to(cv, (_L,))
        return c

    lax.fori_loop(0, b_per_w // _L, addr_body, 0)
    pltpu.async_copy(table_hbm.at[idx_v], rows_v, sem).wait()

    def diag_body(t, c):
        off0 = (base + _L * t) % 128
        dv = plsc.load_gather(rows_v, [t * _L + iota, off0 + iota])
        out_v[pl.ds(t * _L, _L)] = dv
        return c

    lax.fori_loop(0, b_per_w // _L, diag_body, 0)
    pltpu.sync_copy(out_v, out_hbm.at[pl.ds(base, b_per_w)])


def _sc_gather(table128, labels, n):
    b_per_w = n // _NW
    mesh = plsc.VectorSubcoreMesh(core_axis_name="c", subcore_axis_name="s")
    k = pl.kernel(
        _gather_body,
        mesh=mesh,
        out_type=jax.ShapeDtypeStruct((n,), jnp.float32),
        scratch_types=[
            pltpu.VMEM((b_per_w,), jnp.int32),
            pltpu.VMEM((b_per_w,), jnp.int32),
            pltpu.VMEM((b_per_w, 128), jnp.float32),
            pltpu.VMEM((b_per_w,), jnp.float32),
            pltpu.SemaphoreType.DMA,
        ],
        compiler_params=pltpu.CompilerParams(needs_layout_passes=False),
    )
    return k(table128, labels)


def _stats_block(logits_ref, m_ref, z_ref):
    x = logits_ref[...]
    m = jnp.max(x, axis=0, keepdims=True)
    z = jnp.sum(jnp.exp(x - m), axis=0, keepdims=True)
    m_ref[...] = m
    z_ref[...] = z


def _masked_block(logits_ref, labels_ref, xl_ref, m_ref, z_ref, out_ref):
    x = logits_ref[...]
    lab = labels_ref[...]
    xl = xl_ref[...]
    m = m_ref[...]
    z = z_ref[...]
    row = jax.lax.broadcasted_iota(jnp.int32, x.shape, 0)
    e = jnp.exp(x - m)
    mask = (x > xl) | ((x == xl) & (row <= lab))
    num = jnp.sum(jnp.where(mask, e, 0.0), axis=0, keepdims=True)
    out_ref[...] = num / z


@jax.jit
def kernel(logits, labels):
    n, c = logits.shape
    xt = logits.T
    lab1d = labels.astype(jnp.int32)
    table128 = (
        xt.reshape(c // 8, 8, n // 128, 128)
        .transpose(0, 2, 1, 3)
        .reshape(c * n // 128, 128)
    )
    xl = _sc_gather(table128, lab1d, n).reshape(1, n)
    lab2d = lab1d.reshape(1, n)
    bn = _COLS_PER_BLOCK
    row_spec = pl.BlockSpec((1, bn), lambda j: (0, j))
    m, z = pl.pallas_call(
        _stats_block,
        grid=(n // bn,),
        in_specs=[pl.BlockSpec((c, bn), lambda j: (0, j))],
        out_specs=[row_spec, row_spec],
        out_shape=[
            jax.ShapeDtypeStruct((1, n), jnp.float32),
            jax.ShapeDtypeStruct((1, n), jnp.float32),
        ],
        compiler_params=pltpu.CompilerParams(
            dimension_semantics=("parallel",),
        ),
    )(xt)
    out = pl.pallas_call(
        _masked_block,
        grid=(n // bn,),
        in_specs=[
            pl.BlockSpec((c, bn), lambda j: (0, j)),
            row_spec,
            row_spec,
            row_spec,
            row_spec,
        ],
        out_specs=row_spec,
        out_shape=jax.ShapeDtypeStruct((1, n), jnp.float32),
        compiler_params=pltpu.CompilerParams(
            dimension_semantics=("parallel",),
        ),
    )(xt, lab2d, xl, m, z)
    return out.reshape(n)


# speedup vs baseline: 1.2049x; 1.2049x over previous
import functools

import jax
import jax.numpy as jnp
from jax import lax
from jax.experimental import pallas as pl
from jax.experimental.pallas import tpu as pltpu
from jax.experimental.pallas import tpu_sc as plsc


_COLS_PER_BLOCK = 1024
_NC, _NS, _L = 2, 16, 16
_NW = _NC * _NS


def _gather_body(table_hbm, labels_hbm, out_hbm, lab_v, idx_v, rows_v, out_v, sem):
    b_per_w = lab_v.shape[0]
    wid = lax.axis_index("s") * _NC + lax.axis_index("c")
    base = wid * b_per_w
    pltpu.sync_copy(labels_hbm.at[pl.ds(base, b_per_w)], lab_v)
    iota = lax.iota(jnp.int32, _L)

    def addr_body(t, c):
        lv = lab_v[pl.ds(t * _L, _L)]
        cv = (
            lax.shift_left(lax.shift_right_logical(lv, 3), 10)
            + ((base + _L * t) // 128) * 8
            + lax.bitwise_and(lv, 7)
        )
        idx_v[pl.ds(t * _L, _L)] = jnp.broadcast_to(cv, (_L,))
        return c

    lax.fori_loop(0, b_per_w // _L, addr_body, 0)
    pltpu.async_copy(table_hbm.at[idx_v], rows_v, sem).wait()

    def diag_body(t, c):
        off0 = (base + _L * t) % 128
        dv = plsc.load_gather(rows_v, [t * _L + iota, off0 + iota])
        out_v[pl.ds(t * _L, _L)] = dv
        return c

    lax.fori_loop(0, b_per_w // _L, diag_body, 0)
    pltpu.sync_copy(out_v, out_hbm.at[pl.ds(base, b_per_w)])


def _sc_gather(table16, labels, n):
    b_per_w = n // _NW
    mesh = plsc.VectorSubcoreMesh(core_axis_name="c", subcore_axis_name="s")
    k = pl.kernel(
        _gather_body,
        mesh=mesh,
        out_type=jax.ShapeDtypeStruct((n,), jnp.float32),
        scratch_types=[
            pltpu.VMEM((b_per_w,), jnp.int32),
            pltpu.VMEM((b_per_w,), jnp.int32),
            pltpu.VMEM((b_per_w, 128), jnp.float32),
            pltpu.VMEM((b_per_w,), jnp.float32),
            pltpu.SemaphoreType.DMA,
        ],
        compiler_params=pltpu.CompilerParams(needs_layout_passes=False),
    )
    return k(table16, labels)


def _score_block(logits_ref, labels_ref, xl_ref, out_ref):
    x = logits_ref[...]
    lab = labels_ref[...]
    xl = xl_ref[...]
    row = jax.lax.broadcasted_iota(jnp.int32, x.shape, 0)
    t = jnp.minimum(x - (xl + 44.0), 44.0)
    e = jnp.exp(t)
    z = jnp.sum(e, axis=0, keepdims=True)
    mask = (x > xl) | ((x == xl) & (row <= lab))
    num = jnp.sum(jnp.where(mask, e, 0.0), axis=0, keepdims=True)
    out_ref[...] = num / z


@jax.jit
def kernel(logits, labels):
    n, c = logits.shape
    xt = logits.T
    lab1d = labels.astype(jnp.int32)
    table128 = xt.reshape(c // 8, 8, n // 128, 128).transpose(0, 2, 1, 3).reshape(c * n // 128, 128)
    xl = _sc_gather(table128, lab1d, n).reshape(1, n)
    lab2d = lab1d.reshape(1, n)
    bn = _COLS_PER_BLOCK
    out = pl.pallas_call(
        _score_block,
        grid=(n // bn,),
        in_specs=[
            pl.BlockSpec((c, bn), lambda j: (0, j)),
            pl.BlockSpec((1, bn), lambda j: (0, j)),
            pl.BlockSpec((1, bn), lambda j: (0, j)),
        ],
        out_specs=pl.BlockSpec((1, bn), lambda j: (0, j)),
        out_shape=jax.ShapeDtypeStruct((1, n), jnp.float32),
        compiler_params=pltpu.CompilerParams(
            dimension_semantics=("parallel",),
        ),
    )(xt, lab2d, xl)
    return out.reshape(n)


# speedup vs baseline: 1.2276x; 1.0189x over previous
import functools

import jax
import jax.numpy as jnp
from jax import lax
from jax.experimental import pallas as pl
from jax.experimental.pallas import tpu as pltpu
from jax.experimental.pallas import tpu_sc as plsc


_COLS_PER_BLOCK = 512
_NC, _NS, _L = 2, 16, 16
_NW = _NC * _NS


def _gather_body(table_hbm, labels_hbm, out_hbm, lab_v, idx_v, rows_v, out_v, sem):
    b_per_w = lab_v.shape[0]
    wid = lax.axis_index("s") * _NC + lax.axis_index("c")
    base = wid * b_per_w
    pltpu.sync_copy(labels_hbm.at[pl.ds(base, b_per_w)], lab_v)
    iota = lax.iota(jnp.int32, _L)

    def addr_body(t, c):
        lv = lab_v[pl.ds(t * _L, _L)]
        cv = (
            lax.shift_left(lax.shift_right_logical(lv, 3), 10)
            + ((base + _L * t) // 128) * 8
            + lax.bitwise_and(lv, 7)
        )
        idx_v[pl.ds(t * _L, _L)] = jnp.broadcast_to(cv, (_L,))
        return c

    lax.fori_loop(0, b_per_w // _L, addr_body, 0)
    pltpu.async_copy(table_hbm.at[idx_v], rows_v, sem).wait()

    def diag_body(t, c):
        off0 = (base + _L * t) % 128
        dv = plsc.load_gather(rows_v, [t * _L + iota, off0 + iota])
        out_v[pl.ds(t * _L, _L)] = dv
        return c

    lax.fori_loop(0, b_per_w // _L, diag_body, 0)
    pltpu.sync_copy(out_v, out_hbm.at[pl.ds(base, b_per_w)])


def _sc_gather(table16, labels, n):
    b_per_w = n // _NW
    mesh = plsc.VectorSubcoreMesh(core_axis_name="c", subcore_axis_name="s")
    k = pl.kernel(
        _gather_body,
        mesh=mesh,
        out_type=jax.ShapeDtypeStruct((n,), jnp.float32),
        scratch_types=[
            pltpu.VMEM((b_per_w,), jnp.int32),
            pltpu.VMEM((b_per_w,), jnp.int32),
            pltpu.VMEM((b_per_w, 128), jnp.float32),
            pltpu.VMEM((b_per_w,), jnp.float32),
            pltpu.SemaphoreType.DMA,
        ],
        compiler_params=pltpu.CompilerParams(needs_layout_passes=False),
    )
    return k(table16, labels)


def _score_block(logits_ref, labels_ref, xl_ref, out_ref):
    x = logits_ref[...]
    lab = labels_ref[...]
    xl = xl_ref[...]
    row = jax.lax.broadcasted_iota(jnp.int32, x.shape, 0)
    m = jnp.max(x, axis=0, keepdims=True)
    e = jnp.exp(x - m)
    z = jnp.sum(e, axis=0, keepdims=True)
    mask = (x > xl) | ((x == xl) & (row <= lab))
    num = jnp.sum(jnp.where(mask, e, 0.0), axis=0, keepdims=True)
    out_ref[...] = num / z


@jax.jit
def kernel(logits, labels):
    n, c = logits.shape
    xt = logits.T
    lab1d = labels.astype(jnp.int32)
    table128 = xt.reshape(c // 8, 8, n // 128, 128).transpose(0, 2, 1, 3).reshape(c * n // 128, 128)
    xl = _sc_gather(table128, lab1d, n).reshape(1, n)
    lab2d = lab1d.reshape(1, n)
    bn = _COLS_PER_BLOCK
    out = pl.pallas_call(
        _score_block,
        grid=(n // bn,),
        in_specs=[
            pl.BlockSpec((c, bn), lambda j: (0, j)),
            pl.BlockSpec((1, bn), lambda j: (0, j)),
            pl.BlockSpec((1, bn), lambda j: (0, j)),
        ],
        out_specs=pl.BlockSpec((1, bn), lambda j: (0, j)),
        out_shape=jax.ShapeDtypeStruct((1, n), jnp.float32),
        compiler_params=pltpu.CompilerParams(
            dimension_semantics=("parallel",),
        ),
    )(xt, lab2d, xl)
    return out.reshape(n)


# speedup vs baseline: 1.2460x; 1.0150x over previous
import jax
import jax.numpy as jnp
from jax import lax
from jax.experimental import pallas as pl
from jax.experimental.pallas import tpu as pltpu
from jax.experimental.pallas import tpu_sc as plsc


_COLS_PER_BLOCK = 1024
_NC, _NS, _L = 2, 16, 16
_NW = _NC * _NS


def _gather_body(table_hbm, labels_hbm, out_hbm, lab_v, idx_v, rows_v, out_v, sem):
    b_per_w = lab_v.shape[0]
    wid = lax.axis_index("s") * _NC + lax.axis_index("c")
    base = wid * b_per_w
    pltpu.sync_copy(labels_hbm.at[pl.ds(base, b_per_w)], lab_v)
    iota = lax.iota(jnp.int32, _L)

    def addr_body(t, c):
        lv = lab_v[pl.ds(t * _L, _L)]
        cv = (
            lax.shift_left(lax.shift_right_logical(lv, 3), 10)
            + ((base + _L * t) // 128) * 8
            + lax.bitwise_and(lv, 7)
        )
        idx_v[pl.ds(t * _L, _L)] = jnp.broadcast_to(cv, (_L,))
        return c

    lax.fori_loop(0, b_per_w // _L, addr_body, 0)
    pltpu.async_copy(table_hbm.at[idx_v], rows_v, sem).wait()

    def diag_body(t, c):
        off0 = (base + _L * t) % 128
        dv = plsc.load_gather(rows_v, [t * _L + iota, off0 + iota])
        out_v[pl.ds(t * _L, _L)] = dv
        return c

    lax.fori_loop(0, b_per_w // _L, diag_body, 0)
    pltpu.sync_copy(out_v, out_hbm.at[pl.ds(base, b_per_w)])


def _sc_gather(table128, labels, n):
    b_per_w = n // _NW
    mesh = plsc.VectorSubcoreMesh(core_axis_name="c", subcore_axis_name="s")
    k = pl.kernel(
        _gather_body,
        mesh=mesh,
        out_type=jax.ShapeDtypeStruct((n,), jnp.float32),
        scratch_types=[
            pltpu.VMEM((b_per_w,), jnp.int32),
            pltpu.VMEM((b_per_w,), jnp.int32),
            pltpu.VMEM((b_per_w, 128), jnp.float32),
            pltpu.VMEM((b_per_w,), jnp.float32),
            pltpu.SemaphoreType.DMA,
        ],
        compiler_params=pltpu.CompilerParams(needs_layout_passes=False),
    )
    return k(table128, labels)


def _score_block(logits_ref, labels_ref, xl_ref, out_ref):
    x = logits_ref[...]
    lab = labels_ref[...]
    xl = xl_ref[...]
    row = jax.lax.broadcasted_iota(jnp.int32, x.shape, 0)
    m = jnp.max(x, axis=0, keepdims=True)
    e = jnp.exp(x - m)
    z = jnp.sum(e, axis=0, keepdims=True)
    mask = (x > xl) | ((x == xl) & (row <= lab))
    num = jnp.sum(jnp.where(mask, e, 0.0), axis=0, keepdims=True)
    out_ref[...] = num / z


@jax.jit
def kernel(logits, labels):
    n, c = logits.shape
    xt = logits.T
    lab1d = labels.astype(jnp.int32)
    table128 = (
        xt.reshape(c // 8, 8, n // 128, 128)
        .transpose(0, 2, 1, 3)
        .reshape(c * n // 128, 128)
    )
    xl = _sc_gather(table128, lab1d, n).reshape(1, n)
    lab2d = lab1d.reshape(1, n)
    bn = _COLS_PER_BLOCK
    out = pl.pallas_call(
        _score_block,
        grid=(n // bn,),
        in_specs=[
            pl.BlockSpec((c, bn), lambda j: (0, j)),
            pl.BlockSpec((1, bn), lambda j: (0, j)),
            pl.BlockSpec((1, bn), lambda j: (0, j)),
        ],
        out_specs=pl.BlockSpec((1, bn), lambda j: (0, j)),
        out_shape=jax.ShapeDtypeStruct((1, n), jnp.float32),
        compiler_params=pltpu.CompilerParams(
            dimension_semantics=("parallel",),
        ),
    )(xt, lab2d, xl)
    return out.reshape(n)
